# baseline (speedup 1.0000x reference)
import functools

import jax
import jax.numpy as jnp
import numpy as np
from jax.experimental import pallas as pl
from jax.experimental.pallas import tpu as pltpu

_GAMMA = 0.99
_ROWS_PER_BLOCK = 8

_KS0 = np.uint32(0)
_KS1 = np.uint32(42)
_KS2 = np.uint32(np.uint32(0x1BD11BDA) ^ _KS0 ^ _KS1)
_TINY = np.float32(np.finfo(np.float32).tiny)
_SPAN = np.float32(np.float32(1.0) - _TINY)


def _tf_round(x0, x1, r):
    x0 = x0 + x1
    x1 = (x1 << np.uint32(r)) | (x1 >> np.uint32(32 - r))
    x1 = x1 ^ x0
    return x0, x1


def _threefry_bits(flat_idx):
    x0 = jnp.zeros_like(flat_idx) + _KS0
    x1 = flat_idx + _KS1
    rot_a = (13, 15, 26, 6)
    rot_b = (17, 29, 16, 24)
    groups = (rot_a, rot_b, rot_a, rot_b, rot_a)
    inject = ((_KS1, _KS2), (_KS2, _KS0), (_KS0, _KS1), (_KS1, _KS2), (_KS2, _KS0))
    for i in range(5):
        for r in groups[i]:
            x0, x1 = _tf_round(x0, x1, r)
        x0 = x0 + inject[i][0]
        x1 = x1 + inject[i][1] + np.uint32(i + 1)
    return x0 ^ x1


def _gumbel_from_bits(bits):
    fb = (bits >> np.uint32(9)) | np.uint32(0x3F800000)
    f = jax.lax.bitcast_convert_type(fb, jnp.float32) - np.float32(1.0)
    u = jnp.maximum(_TINY, f * _SPAN + _TINY)
    return -jnp.log(-jnp.log(u))


_CHUNK = 512


def _row_pass_kernel(x_ref, lp_ref, act_ref, *, vocab):
    i = pl.program_id(0)
    rows = x_ref.shape[0]
    ch = _CHUNK
    n_full = vocab // ch
    rem = vocab - n_full * ch

    def mx_body(c, m_acc):
        x = x_ref[:, pl.ds(c * ch, ch)]
        return jnp.maximum(m_acc, x)

    m_acc = jax.lax.fori_loop(
        0, n_full, mx_body, jnp.full((rows, ch), -jnp.inf, jnp.float32)
    )
    m = jnp.max(m_acc, axis=-1, keepdims=True)
    if rem:
        xr = x_ref[:, pl.ds(n_full * ch, rem)]
        m = jnp.maximum(m, jnp.max(xr, axis=-1, keepdims=True))

    row_u = jax.lax.broadcasted_iota(jnp.uint32, (rows, ch), 0)
    base = (jnp.uint32(i * rows) + row_u) * jnp.uint32(vocab) + jax.lax.broadcasted_iota(
        jnp.uint32, (rows, ch), 1
    )
    col_i = jax.lax.broadcasted_iota(jnp.int32, (rows, ch), 1)

    def body(c, carry):
        s_acc, bv, bi, bx = carry
        off = c * ch
        x = x_ref[:, pl.ds(off, ch)]
        flat = base + off.astype(jnp.uint32)
        g = _gumbel_from_bits(_threefry_bits(flat))
        y = x + g
        s_acc = s_acc + jnp.exp(x - m)
        col = col_i + off
        upd = y > bv
        bv = jnp.where(upd, y, bv)
        bi = jnp.where(upd, col, bi)
        bx = jnp.where(upd, x, bx)
        return s_acc, bv, bi, bx

    init = (
        jnp.zeros((rows, ch), jnp.float32),
        jnp.full((rows, ch), -jnp.inf, jnp.float32),
        jnp.zeros((rows, ch), jnp.int32),
        jnp.zeros((rows, ch), jnp.float32),
    )
    s_acc, bv, bi, bx = jax.lax.fori_loop(0, n_full, body, init)

    s = jnp.sum(s_acc, axis=-1, keepdims=True)
    v1 = jnp.max(bv, axis=-1, keepdims=True)
    mask1 = bv == v1
    i1 = jnp.min(jnp.where(mask1, bi, jnp.int32(vocab)), axis=-1, keepdims=True)
    x1 = jnp.max(jnp.where(mask1 & (bi == i1), bx, -jnp.inf), axis=-1, keepdims=True)

    if rem:
        off = n_full * ch
        xr = x_ref[:, pl.ds(off, rem)]
        row_u2 = jax.lax.broadcasted_iota(jnp.uint32, (rows, rem), 0)
        flat = (jnp.uint32(i * rows) + row_u2) * jnp.uint32(vocab) + jax.lax.broadcasted_iota(
            jnp.uint32, (rows, rem), 1
        ) + jnp.uint32(off)
        g = _gumbel_from_bits(_threefry_bits(flat))
        y = xr + g
        s = s + jnp.sum(jnp.exp(xr - m), axis=-1, keepdims=True)
        colr = jax.lax.broadcasted_iota(jnp.int32, (rows, rem), 1) + off
        v2 = jnp.max(y, axis=-1, keepdims=True)
        mask2 = y == v2
        i2 = jnp.min(jnp.where(mask2, colr, jnp.int32(vocab)), axis=-1, keepdims=True)
        x2 = jnp.max(
            jnp.where(mask2 & (colr == i2), xr, -jnp.inf), axis=-1, keepdims=True
        )
        use1 = (v1 > v2) | ((v1 == v2) & (i1 < i2))
        i1 = jnp.where(use1, i1, i2)
        x1 = jnp.where(use1, x1, x2)

    lp = (x1 - m) - jnp.log(s)
    lp_ref[...] = lp.reshape(1, rows, 1)
    act_ref[...] = i1.reshape(1, rows, 1)


def _loss_kernel(lp_ref, r_ref, logp_ref, loss_ref, *, steps):
    lp = lp_ref[...]
    logp = jnp.sum(lp, axis=0, keepdims=True)
    r = r_ref[...]
    col = jax.lax.broadcasted_iota(jnp.int32, r.shape, 1)

    def body(i, carry):
        c, disc = carry
        ri = jnp.sum(jnp.where(col == (steps - 1 - i), r, 0.0), keepdims=True)
        c = c * np.float32(_GAMMA) + ri
        disc = jnp.where(col == i, c, disc)
        return (c, disc)

    c0 = jnp.zeros((1, 1), jnp.float32)
    disc0 = jnp.zeros(r.shape, jnp.float32)
    _, disc = jax.lax.fori_loop(0, steps, body, (c0, disc0))

    mean = jnp.sum(disc, keepdims=True) / np.float32(steps)
    centered = disc - mean
    std = jnp.sqrt(jnp.sum(centered * centered, keepdims=True) / np.float32(steps - 1))
    disc_n = centered / std
    loss = jnp.sum(disc_n * (np.float32(-1.0) * logp), keepdims=True)

    logp_ref[...] = logp
    loss_ref[...] = loss


@jax.jit
def kernel(logits, rewards):
    t, b, vocab = logits.shape
    rows = t * b
    rpb = _ROWS_PER_BLOCK
    grid = rows // rpb

    x2 = logits.reshape(rows, vocab)
    lp_blocks, act_blocks = pl.pallas_call(
        functools.partial(_row_pass_kernel, vocab=vocab),
        grid=(grid,),
        in_specs=[pl.BlockSpec((rpb, vocab), lambda i: (i, 0))],
        out_specs=[
            pl.BlockSpec((1, rpb, 1), lambda i: (i, 0, 0)),
            pl.BlockSpec((1, rpb, 1), lambda i: (i, 0, 0)),
        ],
        out_shape=[
            jax.ShapeDtypeStruct((grid, rpb, 1), jnp.float32),
            jax.ShapeDtypeStruct((grid, rpb, 1), jnp.int32),
        ],
        compiler_params=pltpu.CompilerParams(
            dimension_semantics=("arbitrary",),
        ),
    )(x2)

    actions = act_blocks.reshape(t, b)
    lp_tb = lp_blocks.reshape(t, b)

    logp2, loss2 = pl.pallas_call(
        functools.partial(_loss_kernel, steps=t),
        in_specs=[
            pl.BlockSpec((b, t), lambda: (0, 0)),
            pl.BlockSpec((1, t), lambda: (0, 0)),
        ],
        out_specs=[
            pl.BlockSpec((1, t), lambda: (0, 0)),
            pl.BlockSpec((1, 1), lambda: (0, 0)),
        ],
        out_shape=[
            jax.ShapeDtypeStruct((1, t), jnp.float32),
            jax.ShapeDtypeStruct((1, 1), jnp.float32),
        ],
    )(lp_tb.T, rewards.reshape(1, t))

    return actions, logp2[0], loss2[0, 0]


# speedup vs baseline: 1.5796x; 1.5796x over previous
import functools

import jax
import jax.numpy as jnp
import numpy as np
from jax.experimental import pallas as pl
from jax.experimental.pallas import tpu as pltpu

_GAMMA = 0.99
_ROWS_PER_BLOCK = 8

_KS0 = np.uint32(0)
_KS1 = np.uint32(42)
_KS2 = np.uint32(np.uint32(0x1BD11BDA) ^ _KS0 ^ _KS1)
_TINY = np.float32(np.finfo(np.float32).tiny)
_SPAN = np.float32(np.float32(1.0) - _TINY)


def _tf_round(x0, x1, r):
    x0 = x0 + x1
    x1 = (x1 << np.uint32(r)) | (x1 >> np.uint32(32 - r))
    x1 = x1 ^ x0
    return x0, x1


def _threefry_bits(flat_idx):
    x0 = jnp.zeros_like(flat_idx) + _KS0
    x1 = flat_idx + _KS1
    rot_a = (13, 15, 26, 6)
    rot_b = (17, 29, 16, 24)
    groups = (rot_a, rot_b, rot_a, rot_b, rot_a)
    inject = ((_KS1, _KS2), (_KS2, _KS0), (_KS0, _KS1), (_KS1, _KS2), (_KS2, _KS0))
    for i in range(5):
        for r in groups[i]:
            x0, x1 = _tf_round(x0, x1, r)
        x0 = x0 + inject[i][0]
        x1 = x1 + inject[i][1] + np.uint32(i + 1)
    return x0 ^ x1


def _gumbel_from_bits(bits):
    fb = (bits >> np.uint32(9)) | np.uint32(0x3F800000)
    f = jax.lax.bitcast_convert_type(fb, jnp.float32) - np.float32(1.0)
    u = jnp.maximum(_TINY, f * _SPAN + _TINY)
    return -jnp.log(-jnp.log(u))


_CHUNK = 1024


def _row_pass_kernel(x_ref, lp_ref, act_ref, *, vocab):
    i = pl.program_id(0)
    rows = x_ref.shape[0]
    ch = _CHUNK
    n_full = vocab // ch
    rem = vocab - n_full * ch

    def mx_body(c, m_acc):
        x = x_ref[:, pl.ds(c * ch, ch)]
        return jnp.maximum(m_acc, x)

    m_acc = jax.lax.fori_loop(
        0, n_full, mx_body, jnp.full((rows, ch), -jnp.inf, jnp.float32)
    )
    m = jnp.max(m_acc, axis=-1, keepdims=True)
    if rem:
        xr = x_ref[:, pl.ds(n_full * ch, rem)]
        m = jnp.maximum(m, jnp.max(xr, axis=-1, keepdims=True))

    row_u = jax.lax.broadcasted_iota(jnp.uint32, (rows, ch), 0)
    base = (jnp.uint32(i * rows) + row_u) * jnp.uint32(vocab) + jax.lax.broadcasted_iota(
        jnp.uint32, (rows, ch), 1
    )
    col_i = jax.lax.broadcasted_iota(jnp.int32, (rows, ch), 1)

    def body(c, carry):
        s_acc, bv, bi, bx = carry
        off = c * ch
        x = x_ref[:, pl.ds(off, ch)]
        flat = base + off.astype(jnp.uint32)
        g = _gumbel_from_bits(_threefry_bits(flat))
        y = x + g
        s_acc = s_acc + jnp.exp(x - m)
        col = col_i + off
        upd = y > bv
        bv = jnp.where(upd, y, bv)
        bi = jnp.where(upd, col, bi)
        bx = jnp.where(upd, x, bx)
        return s_acc, bv, bi, bx

    init = (
        jnp.zeros((rows, ch), jnp.float32),
        jnp.full((rows, ch), -jnp.inf, jnp.float32),
        jnp.zeros((rows, ch), jnp.int32),
        jnp.zeros((rows, ch), jnp.float32),
    )
    s_acc, bv, bi, bx = jax.lax.fori_loop(0, n_full, body, init)

    s = jnp.sum(s_acc, axis=-1, keepdims=True)
    v1 = jnp.max(bv, axis=-1, keepdims=True)
    mask1 = bv == v1
    i1 = jnp.min(jnp.where(mask1, bi, jnp.int32(vocab)), axis=-1, keepdims=True)
    x1 = jnp.max(jnp.where(mask1 & (bi == i1), bx, -jnp.inf), axis=-1, keepdims=True)

    if rem:
        off = n_full * ch
        xr = x_ref[:, pl.ds(off, rem)]
        row_u2 = jax.lax.broadcasted_iota(jnp.uint32, (rows, rem), 0)
        flat = (jnp.uint32(i * rows) + row_u2) * jnp.uint32(vocab) + jax.lax.broadcasted_iota(
            jnp.uint32, (rows, rem), 1
        ) + jnp.uint32(off)
        g = _gumbel_from_bits(_threefry_bits(flat))
        y = xr + g
        s = s + jnp.sum(jnp.exp(xr - m), axis=-1, keepdims=True)
        colr = jax.lax.broadcasted_iota(jnp.int32, (rows, rem), 1) + off
        v2 = jnp.max(y, axis=-1, keepdims=True)
        mask2 = y == v2
        i2 = jnp.min(jnp.where(mask2, colr, jnp.int32(vocab)), axis=-1, keepdims=True)
        x2 = jnp.max(
            jnp.where(mask2 & (colr == i2), xr, -jnp.inf), axis=-1, keepdims=True
        )
        use1 = (v1 > v2) | ((v1 == v2) & (i1 < i2))
        i1 = jnp.where(use1, i1, i2)
        x1 = jnp.where(use1, x1, x2)

    lp = (x1 - m) - jnp.log(s)
    lp_ref[...] = lp.reshape(1, rows, 1)
    act_ref[...] = i1.reshape(1, rows, 1)


def _loss_kernel(lp_ref, r_ref, logp_ref, loss_ref, *, steps):
    lp = lp_ref[...]
    logp = jnp.sum(lp, axis=0, keepdims=True)
    r = r_ref[...]
    col = jax.lax.broadcasted_iota(jnp.int32, r.shape, 1)

    def body(i, carry):
        c, disc = carry
        ri = jnp.sum(jnp.where(col == (steps - 1 - i), r, 0.0), keepdims=True)
        c = c * np.float32(_GAMMA) + ri
        disc = jnp.where(col == i, c, disc)
        return (c, disc)

    c0 = jnp.zeros((1, 1), jnp.float32)
    disc0 = jnp.zeros(r.shape, jnp.float32)
    _, disc = jax.lax.fori_loop(0, steps, body, (c0, disc0))

    mean = jnp.sum(disc, keepdims=True) / np.float32(steps)
    centered = disc - mean
    std = jnp.sqrt(jnp.sum(centered * centered, keepdims=True) / np.float32(steps - 1))
    disc_n = centered / std
    loss = jnp.sum(disc_n * (np.float32(-1.0) * logp), keepdims=True)

    logp_ref[...] = logp
    loss_ref[...] = loss


@jax.jit
def kernel(logits, rewards):
    t, b, vocab = logits.shape
    rows = t * b
    rpb = _ROWS_PER_BLOCK
    grid = rows // rpb

    x2 = logits.reshape(rows, vocab)
    lp_blocks, act_blocks = pl.pallas_call(
        functools.partial(_row_pass_kernel, vocab=vocab),
        grid=(grid,),
        in_specs=[pl.BlockSpec((rpb, vocab), lambda i: (i, 0))],
        out_specs=[
            pl.BlockSpec((1, rpb, 1), lambda i: (i, 0, 0)),
            pl.BlockSpec((1, rpb, 1), lambda i: (i, 0, 0)),
        ],
        out_shape=[
            jax.ShapeDtypeStruct((grid, rpb, 1), jnp.float32),
            jax.ShapeDtypeStruct((grid, rpb, 1), jnp.int32),
        ],
        compiler_params=pltpu.CompilerParams(
            dimension_semantics=("arbitrary",),
        ),
    )(x2)

    actions = act_blocks.reshape(t, b)
    lp_tb = lp_blocks.reshape(t, b)

    logp2, loss2 = pl.pallas_call(
        functools.partial(_loss_kernel, steps=t),
        in_specs=[
            pl.BlockSpec((b, t), lambda: (0, 0)),
            pl.BlockSpec((1, t), lambda: (0, 0)),
        ],
        out_specs=[
            pl.BlockSpec((1, t), lambda: (0, 0)),
            pl.BlockSpec((1, 1), lambda: (0, 0)),
        ],
        out_shape=[
            jax.ShapeDtypeStruct((1, t), jnp.float32),
            jax.ShapeDtypeStruct((1, 1), jnp.float32),
        ],
    )(lp_tb.T, rewards.reshape(1, t))

    return actions, logp2[0], loss2[0, 0]
